# baseline (speedup 1.0000x reference)
import functools

import numpy as np
import jax
import jax.numpy as jnp
from jax.experimental import pallas as pl
from jax.experimental.pallas import tpu as pltpu

_VGG19_STD = np.asarray([0.229, 0.224, 0.225], dtype=np.float32)
_W_C = (np.float32(0.25) / (_VGG19_STD * _VGG19_STD)).astype(np.float32)


def _fold_hw(v, h, w):
    s = v[..., 0:128]
    for j in range(1, w // 128):
        s = s + v[..., j * 128:(j + 1) * 128]
    t = s[:, 0:8, :]
    for j in range(1, h // 8):
        t = t + s[:, j * 8:(j + 1) * 8, :]
    u = t[0]
    for j in range(1, t.shape[0]):
        u = u + t[j]
    return u


def _wsq_kernel(x_ref, y_ref, o_ref, *, h, w, weights):
    d = x_ref[...] - y_ref[...]
    c2 = d * d
    acc = None
    for c, wc in enumerate(weights):
        part = _fold_hw(c2[:, c], h, w) * wc
        acc = part if acc is None else acc + part
    o_ref[...] = acc


def kernel(x, y):
    n, c_in, h, w = x.shape

    if c_in == 3:
        weights = (float(_W_C[0]), float(_W_C[1]), float(_W_C[2]))
    else:
        weights = (float(np.float32(_W_C[0] + _W_C[1] + _W_C[2])),)

    bn = 1
    for cand in (2, 4, 1):
        if n % cand == 0 and (n // cand) % 2 == 0:
            bn = cand
            break
    grid = n // bn

    block_in = bn * c_in * h * w * 4
    vmem_limit = int(min(2 * 2 * block_in + (4 << 20), 60 << 20))

    body = functools.partial(_wsq_kernel, h=h, w=w, weights=weights)
    partials = pl.pallas_call(
        body,
        out_shape=jax.ShapeDtypeStruct((grid * 8, 128), jnp.float32),
        grid=(grid,),
        in_specs=[
            pl.BlockSpec((bn, c_in, h, w), lambda i: (i, 0, 0, 0)),
            pl.BlockSpec((bn, c_in, h, w), lambda i: (i, 0, 0, 0)),
        ],
        out_specs=pl.BlockSpec((8, 128), lambda i: (i, 0)),
        compiler_params=pltpu.CompilerParams(
            dimension_semantics=("parallel",),
            vmem_limit_bytes=vmem_limit,
        ),
        cost_estimate=pl.CostEstimate(
            flops=3 * n * c_in * h * w,
            transcendentals=0,
            bytes_accessed=2 * n * c_in * h * w * 4 + grid * 8 * 128 * 4,
        ),
    )(x, y)

    scale = np.float32(1.0) / (np.float32(3.0) * np.float32(h * w) * np.float32(n))
    return jnp.sum(partials) * scale


# speedup vs baseline: 1.1520x; 1.1520x over previous
import functools

import numpy as np
import jax
import jax.numpy as jnp
from jax.experimental import pallas as pl
from jax.experimental.pallas import tpu as pltpu

_VGG19_STD = np.asarray([0.229, 0.224, 0.225], dtype=np.float32)
_W_C = (np.float32(0.25) / (_VGG19_STD * _VGG19_STD)).astype(np.float32)


def _fold_hw(v, h, w):
    s = v[..., 0:128]
    for j in range(1, w // 128):
        s = s + v[..., j * 128:(j + 1) * 128]
    t = s[:, 0:8, :]
    for j in range(1, h // 8):
        t = t + s[:, j * 8:(j + 1) * 8, :]
    u = t[0]
    for j in range(1, t.shape[0]):
        u = u + t[j]
    return u


def _wsq_kernel(x_ref, y_ref, o_ref, *, h, w, weights):
    d = x_ref[...] - y_ref[...]
    c2 = d * d
    acc = None
    for c, wc in enumerate(weights):
        part = _fold_hw(c2[:, c], h, w) * wc
        acc = part if acc is None else acc + part
    o_ref[...] = acc


def kernel(x, y):
    n, c_in, h, w = x.shape

    if c_in == 3:
        weights = (float(_W_C[0]), float(_W_C[1]), float(_W_C[2]))
    else:
        weights = (float(np.float32(_W_C[0] + _W_C[1] + _W_C[2])),)

    bn = 1
    for cand in (8, 4, 2, 1):
        if n % cand == 0 and (n // cand) % 2 == 0:
            bn = cand
            break
    grid = n // bn

    block_in = bn * c_in * h * w * 4
    vmem_limit = int(min(2 * 2 * block_in + (8 << 20), 60 << 20))

    body = functools.partial(_wsq_kernel, h=h, w=w, weights=weights)
    partials = pl.pallas_call(
        body,
        out_shape=jax.ShapeDtypeStruct((grid * 8, 128), jnp.float32),
        grid=(grid,),
        in_specs=[
            pl.BlockSpec((bn, c_in, h, w), lambda i: (i, 0, 0, 0)),
            pl.BlockSpec((bn, c_in, h, w), lambda i: (i, 0, 0, 0)),
        ],
        out_specs=pl.BlockSpec((8, 128), lambda i: (i, 0)),
        compiler_params=pltpu.CompilerParams(
            dimension_semantics=("parallel",),
            vmem_limit_bytes=vmem_limit,
        ),
        cost_estimate=pl.CostEstimate(
            flops=3 * n * c_in * h * w,
            transcendentals=0,
            bytes_accessed=2 * n * c_in * h * w * 4 + grid * 8 * 128 * 4,
        ),
    )(x, y)

    scale = np.float32(1.0) / (np.float32(3.0) * np.float32(h * w) * np.float32(n))
    return jnp.sum(partials) * scale


# speedup vs baseline: 1.1881x; 1.0313x over previous
import functools

import numpy as np
import jax
import jax.numpy as jnp
from jax.experimental import pallas as pl
from jax.experimental.pallas import tpu as pltpu

_VGG19_STD = np.asarray([0.229, 0.224, 0.225], dtype=np.float32)
_W_C = (np.float32(0.25) / (_VGG19_STD * _VGG19_STD)).astype(np.float32)


def _fold_hw(v, h, w):
    s = v[..., 0:128]
    for j in range(1, w // 128):
        s = s + v[..., j * 128:(j + 1) * 128]
    t = s[:, 0:8, :]
    for j in range(1, h // 8):
        t = t + s[:, j * 8:(j + 1) * 8, :]
    u = t[0]
    for j in range(1, t.shape[0]):
        u = u + t[j]
    return u


def _wsq_kernel(x_ref, y_ref, o_ref, *, h, w, weights):
    d = x_ref[...] - y_ref[...]
    c2 = d * d
    acc = None
    for c, wc in enumerate(weights):
        part = _fold_hw(c2[:, c], h, w) * wc
        acc = part if acc is None else acc + part
    o_ref[...] = acc


def kernel(x, y):
    n, c_in, h, w = x.shape

    if c_in == 3:
        weights = (float(_W_C[0]), float(_W_C[1]), float(_W_C[2]))
    else:
        weights = (float(np.float32(_W_C[0] + _W_C[1] + _W_C[2])),)

    bn = 1
    for cand in (4, 2, 1):
        if n % cand == 0 and (n // cand) % 2 == 0:
            bn = cand
            break
    grid = n // bn

    block_in = bn * c_in * h * w * 4
    vmem_limit = int(min(2 * 2 * block_in + (8 << 20), 60 << 20))

    body = functools.partial(_wsq_kernel, h=h, w=w, weights=weights)
    partials = pl.pallas_call(
        body,
        out_shape=jax.ShapeDtypeStruct((grid * 8, 128), jnp.float32),
        grid=(grid,),
        in_specs=[
            pl.BlockSpec((bn, c_in, h, w), lambda i: (i, 0, 0, 0)),
            pl.BlockSpec((bn, c_in, h, w), lambda i: (i, 0, 0, 0)),
        ],
        out_specs=pl.BlockSpec((8, 128), lambda i: (i, 0)),
        compiler_params=pltpu.CompilerParams(
            dimension_semantics=("parallel",),
            vmem_limit_bytes=vmem_limit,
        ),
        cost_estimate=pl.CostEstimate(
            flops=3 * n * c_in * h * w,
            transcendentals=0,
            bytes_accessed=2 * n * c_in * h * w * 4 + grid * 8 * 128 * 4,
        ),
    )(x, y)

    scale = np.float32(1.0) / (np.float32(3.0) * np.float32(h * w) * np.float32(n))
    return jnp.sum(partials) * scale
